# Initial kernel scaffold; baseline (speedup 1.0000x reference)
import jax
import jax.numpy as jnp
from jax.experimental import pallas as pl


def kernel(x, pos, edge_attr, edge_index, params):
    raise NotImplementedError("write your pallas kernel here")


# speedup vs baseline: 1.6700x; 1.6700x over previous
import functools

import jax
import jax.numpy as jnp
from jax import lax
from jax.experimental import pallas as pl
from jax.experimental.pallas import tpu as pltpu
from jax.experimental.pallas import tpu_sc as plsc

N = 10000
E = 320000
LATENT = 256
HALF = 128
NUM_CT = 20


_ROWS = 2000


def _enc_body(xc, w1, b1, w2, b2, w3, b3, out):
    h1 = jnp.maximum(jnp.dot(xc[...], w1[...], preferred_element_type=jnp.float32) + b1[...], 0.0)
    h2 = jnp.maximum(jnp.dot(h1, w2[...], preferred_element_type=jnp.float32) + b2[...], 0.0)
    out[...] = jnp.dot(h2, w3[...], preferred_element_type=jnp.float32) + b3[...]


def _encoder(xc, p):
    grid = N // _ROWS
    full = lambda shape: pl.BlockSpec(shape, lambda i: (0, 0))
    return pl.pallas_call(
        _enc_body,
        grid=(grid,),
        in_specs=[
            pl.BlockSpec((_ROWS, 130), lambda i: (i, 0)),
            full((130, 512)), full((1, 512)),
            full((512, 256)), full((1, 256)),
            full((256, LATENT)), full((1, LATENT)),
        ],
        out_specs=pl.BlockSpec((_ROWS, LATENT), lambda i: (i, 0)),
        out_shape=jax.ShapeDtypeStruct((N, LATENT), jnp.float32),
    )(xc, p["mlp_W1"], p["mlp_b1"].reshape(1, -1), p["mlp_W2"],
      p["mlp_b2"].reshape(1, -1), p["mlp_W3"], p["mlp_b3"].reshape(1, -1))


def _xlxr_body(h, wl, bl, wr, br, xl0, xl1, xr0, xr1):
    xl = jnp.dot(h[...], wl[...], preferred_element_type=jnp.float32) + bl[...]
    xr = jnp.dot(h[...], wr[...], preferred_element_type=jnp.float32) + br[...]
    xl0[...] = xl[:, :HALF]
    xl1[...] = xl[:, HALF:]
    xr0[...] = xr[:, :HALF]
    xr1[...] = xr[:, HALF:]


def _xlxr(h, bp):
    grid = N // _ROWS
    full = lambda shape: pl.BlockSpec(shape, lambda i: (0, 0))
    outspec = pl.BlockSpec((_ROWS, HALF), lambda i: (i, 0))
    oshape = jax.ShapeDtypeStruct((N, HALF), jnp.float32)
    return pl.pallas_call(
        _xlxr_body,
        grid=(grid,),
        in_specs=[
            pl.BlockSpec((_ROWS, LATENT), lambda i: (i, 0)),
            full((LATENT, LATENT)), full((1, LATENT)),
            full((LATENT, LATENT)), full((1, LATENT)),
        ],
        out_specs=[outspec, outspec, outspec, outspec],
        out_shape=[oshape, oshape, oshape, oshape],
    )(h, bp["Wl"], bp["bl"].reshape(1, -1), bp["Wr"], bp["br"].reshape(1, -1))


def _post_body(h, s0, s1, den, gb, wc, bc, g1, b1, wf1, bf1, wf2, bf2, g2, b2, out):
    S = jnp.concatenate([s0[...], s1[...]], axis=-1)
    osp = S / (den[...] + 1e-16) + gb[...]
    o = h[...] + jnp.dot(osp, wc[...], preferred_element_type=jnp.float32) + bc[...]
    m1 = jnp.mean(o, axis=0, keepdims=True)
    v1 = jnp.mean((o - m1) * (o - m1), axis=0, keepdims=True)
    o = (o - m1) * jax.lax.rsqrt(v1 + 1e-5) * g1[...] + b1[...]
    hh = jnp.maximum(jnp.dot(o, wf1[...], preferred_element_type=jnp.float32) + bf1[...], 0.0)
    ffn = jnp.dot(hh, wf2[...], preferred_element_type=jnp.float32) + bf2[...]
    o2 = o + ffn
    m2 = jnp.mean(o2, axis=0, keepdims=True)
    v2 = jnp.mean((o2 - m2) * (o2 - m2), axis=0, keepdims=True)
    out[...] = (o2 - m2) * jax.lax.rsqrt(v2 + 1e-5) * g2[...] + b2[...]


def _post(h, s0, s1, den, bp):
    return pl.pallas_call(
        _post_body,
        out_shape=jax.ShapeDtypeStruct((N, LATENT), jnp.float32),
    )(h, s0, s1, den.reshape(N, 1), bp["gat_bias"].reshape(1, -1),
      bp["Wc"], bp["bc"].reshape(1, -1),
      bp["bn1_g"].reshape(1, -1), bp["bn1_b"].reshape(1, -1),
      bp["Wf1"], bp["bf1"].reshape(1, -1), bp["Wf2"], bp["bf2"].reshape(1, -1),
      bp["bn2_g"].reshape(1, -1), bp["bn2_b"].reshape(1, -1))


def _dec_body(h, wd, bd, out):
    logits = jnp.dot(h[...], wd[...], preferred_element_type=jnp.float32) + bd[...]
    m = jnp.max(logits, axis=-1, keepdims=True)
    e = jnp.exp(logits - m)
    out[...] = e / jnp.sum(e, axis=-1, keepdims=True)


def _decoder(h, p):
    wd = jnp.pad(p["dec_W"], ((0, 0), (0, HALF - NUM_CT)))
    bd = jnp.pad(p["dec_b"], (0, HALF - NUM_CT), constant_values=-1e30)
    probs = pl.pallas_call(
        _dec_body,
        grid=(N // _ROWS,),
        in_specs=[
            pl.BlockSpec((_ROWS, LATENT), lambda i: (i, 0)),
            pl.BlockSpec((LATENT, HALF), lambda i: (0, 0)),
            pl.BlockSpec((1, HALF), lambda i: (0, 0)),
        ],
        out_specs=pl.BlockSpec((_ROWS, HALF), lambda i: (i, 0)),
        out_shape=jax.ShapeDtypeStruct((N, HALF), jnp.float32),
    )(h, wd, bd.reshape(1, -1))
    return probs[:, :NUM_CT]


_CH = 80
_EPT_A = E // 32
_EPT_B = E // 16


def _sc_mesh():
    return plsc.VectorSubcoreMesh(core_axis_name="c", subcore_axis_name="s")


_SC_PARAMS = pltpu.CompilerParams(needs_layout_passes=False)


def _pass_a(xl0, xl1, xr0, xr1, src, dst, ea, we, att):
    @functools.partial(
        pl.kernel,
        out_type=jax.ShapeDtypeStruct((E,), jnp.float32),
        mesh=_sc_mesh(),
        scratch_types=[
            pltpu.VMEM((_CH,), jnp.int32),
            pltpu.VMEM((_CH,), jnp.int32),
            pltpu.VMEM((_CH, HALF), jnp.float32),
            pltpu.VMEM((_CH, HALF), jnp.float32),
            pltpu.VMEM((_CH, HALF), jnp.float32),
            pltpu.VMEM((_CH, HALF), jnp.float32),
            pltpu.VMEM((_CH,), jnp.float32),
            pltpu.VMEM((_CH,), jnp.float32),
            pltpu.VMEM((LATENT,), jnp.float32),
            pltpu.VMEM((LATENT,), jnp.float32),
            pltpu.SemaphoreType.DMA,
        ],
        compiler_params=_SC_PARAMS,
    )
    def k(xl0h, xl1h, xr0h, xr1h, srch, dsth, eah, weh, atth, exh,
          src_v, dst_v, a0, a1, b0, b1, ea_v, ex_v, we_v, att_v, sem):
        cid = lax.axis_index("c")
        sid = lax.axis_index("s")
        wid = sid * 2 + cid
        base = wid * _EPT_A
        pltpu.sync_copy(weh, we_v)
        pltpu.sync_copy(atth, att_v)

        def chunk(j, carry):
            off = base + j * _CH
            pltpu.sync_copy(srch.at[pl.ds(off, _CH)], src_v)
            pltpu.sync_copy(dsth.at[pl.ds(off, _CH)], dst_v)
            pltpu.sync_copy(eah.at[pl.ds(off, _CH)], ea_v)
            cps = [
                pltpu.async_copy(xl0h.at[src_v], a0, sem),
                pltpu.async_copy(xl1h.at[src_v], a1, sem),
                pltpu.async_copy(xr0h.at[dst_v], b0, sem),
                pltpu.async_copy(xr1h.at[dst_v], b1, sem),
            ]
            for cp in cps:
                cp.wait()
            for g in range(_CH // 16):
                e16 = lax.iota(jnp.int32, 16) + (g * 16)
                eav = ea_v[pl.ds(g * 16, 16)]

                def feat(f, acc):
                    fb = jnp.zeros((16,), jnp.int32) + f
                    wef0 = plsc.load_gather(we_v, [fb])
                    atf0 = plsc.load_gather(att_v, [fb])
                    v0 = (plsc.load_gather(a0, [e16, fb])
                          + plsc.load_gather(b0, [e16, fb])
                          + eav * wef0)
                    acc = acc + jnp.maximum(v0, 0.2 * v0) * atf0
                    fb1 = fb + HALF
                    wef1 = plsc.load_gather(we_v, [fb1])
                    atf1 = plsc.load_gather(att_v, [fb1])
                    v1 = (plsc.load_gather(a1, [e16, fb])
                          + plsc.load_gather(b1, [e16, fb])
                          + eav * wef1)
                    return acc + jnp.maximum(v1, 0.2 * v1) * atf1

                acc = lax.fori_loop(0, HALF, feat, jnp.zeros((16,), jnp.float32))
                ex_v[pl.ds(g * 16, 16)] = jnp.exp(acc)
            pltpu.sync_copy(ex_v, exh.at[pl.ds(off, _CH)])
            return carry

        lax.fori_loop(0, _EPT_A // _CH, chunk, 0)

    return k(xl0, xl1, xr0, xr1, src, dst, ea, we, att)


def _pass_b(xl0, xl1, src, dst, ex):
    oshape = jax.ShapeDtypeStruct((N, HALF), jnp.float32)
    @functools.partial(
        pl.kernel,
        out_type=[oshape, oshape, jax.ShapeDtypeStruct((N,), jnp.float32)],
        mesh=_sc_mesh(),
        scratch_types=[
            pltpu.VMEM((_CH,), jnp.int32),
            pltpu.VMEM((_CH,), jnp.int32),
            pltpu.VMEM((_CH, HALF), jnp.float32),
            pltpu.VMEM((_CH,), jnp.float32),
            pltpu.VMEM((80, HALF), jnp.float32),
            pltpu.VMEM((1008,), jnp.float32),
            pltpu.VMEM_SHARED((N, HALF), jnp.float32),
            pltpu.VMEM_SHARED((N,), jnp.float32),
            pltpu.SemaphoreType.DMA,
        ],
        compiler_params=_SC_PARAMS,
    )
    def k(xl0h, xl1h, srch, dsth, exh, s0h, s1h, denh,
          src_v, dst_v, rows_v, ex_v, zbuf, zden, s_sh, den_sh, sem):
        cid = lax.axis_index("c")
        sid = lax.axis_index("s")
        zv = jnp.zeros((16,), jnp.float32)

        def zrow(i, c):
            for cc in range(HALF // 16):
                zbuf[i, pl.ds(cc * 16, 16)] = zv
            return c

        lax.fori_loop(0, 80, zrow, 0)

        def zel(i, c):
            zden[pl.ds(i * 16, 16)] = zv
            return c

        lax.fori_loop(0, 63, zel, 0)

        for q in range(8):
            bid = sid + q * 16

            @pl.when(bid < 125)
            def _():
                pltpu.sync_copy(zbuf, s_sh.at[pl.ds(bid * 80, 80)])

        @pl.when(sid < 10)
        def _():
            pltpu.sync_copy(zden.at[pl.ds(0, 1000)], den_sh.at[pl.ds(sid * 1000, 1000)])

        plsc.subcore_barrier()

        base = sid * _EPT_B

        def chunk(j, carry):
            off = base + j * _CH
            pltpu.sync_copy(srch.at[pl.ds(off, _CH)], src_v)
            pltpu.sync_copy(dsth.at[pl.ds(off, _CH)], dst_v)
            pltpu.sync_copy(exh.at[pl.ds(off, _CH)], ex_v)

            @pl.when(cid == 0)
            def _():
                pltpu.async_copy(xl0h.at[src_v], rows_v, sem).wait()

            @pl.when(cid == 1)
            def _():
                pltpu.async_copy(xl1h.at[src_v], rows_v, sem).wait()

            def scale(e, c2):
                s = plsc.load_gather(ex_v, [jnp.zeros((16,), jnp.int32) + e])
                for cc in range(HALF // 16):
                    rows_v[e, pl.ds(cc * 16, 16)] = rows_v[e, pl.ds(cc * 16, 16)] * s
                return c2

            lax.fori_loop(0, _CH, scale, 0)
            pltpu.sync_copy(rows_v, s_sh.at[dst_v], add=True)

            @pl.when(cid == 0)
            def _():
                pltpu.sync_copy(ex_v, den_sh.at[dst_v], add=True)

            return carry

        lax.fori_loop(0, _EPT_B // _CH, chunk, 0)
        plsc.subcore_barrier()

        for q in range(8):
            bid = sid + q * 16

            @pl.when((bid < 125) & (cid == 0))
            def _():
                sl = pl.ds(bid * 80, 80)
                pltpu.sync_copy(s_sh.at[sl], zbuf)
                pltpu.sync_copy(zbuf, s0h.at[sl])

            @pl.when((bid < 125) & (cid == 1))
            def _():
                sl = pl.ds(bid * 80, 80)
                pltpu.sync_copy(s_sh.at[sl], zbuf)
                pltpu.sync_copy(zbuf, s1h.at[sl])

        @pl.when((cid == 0) & (sid < 10))
        def _():
            pltpu.sync_copy(den_sh.at[pl.ds(sid * 1000, 1000)],
                            zden.at[pl.ds(0, 1000)])
            pltpu.sync_copy(zden.at[pl.ds(0, 1000)],
                            denh.at[pl.ds(sid * 1000, 1000)])

    return k(xl0, xl1, src, dst, ex)


def kernel(x, pos, edge_attr, edge_index, params):
    src = edge_index[0]
    dst = edge_index[1]
    ea = edge_attr[:, 0]
    xc = jnp.concatenate([x, pos], axis=-1)
    h = _encoder(xc, params)
    for bp in params["blocks"]:
        xl0, xl1, xr0, xr1 = _xlxr(h, bp)
        ex = _pass_a(xl0, xl1, xr0, xr1, src, dst, ea,
                     bp["We"][0], bp["att"])
        s0, s1, den = _pass_b(xl0, xl1, src, dst, ex)
        h = _post(h, s0, s1, den, bp)
    return _decoder(h, params)


# speedup vs baseline: 1.7588x; 1.0532x over previous
import functools

import jax
import jax.numpy as jnp
from jax import lax
from jax.experimental import pallas as pl
from jax.experimental.pallas import tpu as pltpu
from jax.experimental.pallas import tpu_sc as plsc

N = 10000
E = 320000
LATENT = 256
HALF = 128
NUM_CT = 20


_ROWS = 2000


def _enc_body(xc, w1, b1, w2, b2, w3, b3, out):
    h1 = jnp.maximum(jnp.dot(xc[...], w1[...], preferred_element_type=jnp.float32) + b1[...], 0.0)
    h2 = jnp.maximum(jnp.dot(h1, w2[...], preferred_element_type=jnp.float32) + b2[...], 0.0)
    out[...] = jnp.dot(h2, w3[...], preferred_element_type=jnp.float32) + b3[...]


def _encoder(xc, p):
    grid = N // _ROWS
    full = lambda shape: pl.BlockSpec(shape, lambda i: (0, 0))
    return pl.pallas_call(
        _enc_body,
        grid=(grid,),
        in_specs=[
            pl.BlockSpec((_ROWS, 130), lambda i: (i, 0)),
            full((130, 512)), full((1, 512)),
            full((512, 256)), full((1, 256)),
            full((256, LATENT)), full((1, LATENT)),
        ],
        out_specs=pl.BlockSpec((_ROWS, LATENT), lambda i: (i, 0)),
        out_shape=jax.ShapeDtypeStruct((N, LATENT), jnp.float32),
    )(xc, p["mlp_W1"], p["mlp_b1"].reshape(1, -1), p["mlp_W2"],
      p["mlp_b2"].reshape(1, -1), p["mlp_W3"], p["mlp_b3"].reshape(1, -1))


def _xlxr_body(h, wl, bl, wr, br, xl0, xl1, xr0, xr1):
    xl = jnp.dot(h[...], wl[...], preferred_element_type=jnp.float32) + bl[...]
    xr = jnp.dot(h[...], wr[...], preferred_element_type=jnp.float32) + br[...]
    xl0[...] = xl[:, :HALF]
    xl1[...] = xl[:, HALF:]
    xr0[...] = xr[:, :HALF]
    xr1[...] = xr[:, HALF:]


def _xlxr(h, bp):
    grid = N // _ROWS
    full = lambda shape: pl.BlockSpec(shape, lambda i: (0, 0))
    outspec = pl.BlockSpec((_ROWS, HALF), lambda i: (i, 0))
    oshape = jax.ShapeDtypeStruct((N, HALF), jnp.float32)
    return pl.pallas_call(
        _xlxr_body,
        grid=(grid,),
        in_specs=[
            pl.BlockSpec((_ROWS, LATENT), lambda i: (i, 0)),
            full((LATENT, LATENT)), full((1, LATENT)),
            full((LATENT, LATENT)), full((1, LATENT)),
        ],
        out_specs=[outspec, outspec, outspec, outspec],
        out_shape=[oshape, oshape, oshape, oshape],
    )(h, bp["Wl"], bp["bl"].reshape(1, -1), bp["Wr"], bp["br"].reshape(1, -1))


def _post_body(h, s0, s1, den, gb, wc, bc, g1, b1, wf1, bf1, wf2, bf2, g2, b2, out):
    S = jnp.concatenate([s0[...], s1[...]], axis=-1)
    osp = S / (den[...] + 1e-16) + gb[...]
    o = h[...] + jnp.dot(osp, wc[...], preferred_element_type=jnp.float32) + bc[...]
    m1 = jnp.mean(o, axis=0, keepdims=True)
    v1 = jnp.mean((o - m1) * (o - m1), axis=0, keepdims=True)
    o = (o - m1) * jax.lax.rsqrt(v1 + 1e-5) * g1[...] + b1[...]
    hh = jnp.maximum(jnp.dot(o, wf1[...], preferred_element_type=jnp.float32) + bf1[...], 0.0)
    ffn = jnp.dot(hh, wf2[...], preferred_element_type=jnp.float32) + bf2[...]
    o2 = o + ffn
    m2 = jnp.mean(o2, axis=0, keepdims=True)
    v2 = jnp.mean((o2 - m2) * (o2 - m2), axis=0, keepdims=True)
    out[...] = (o2 - m2) * jax.lax.rsqrt(v2 + 1e-5) * g2[...] + b2[...]


def _post(h, s0, s1, den, bp):
    return pl.pallas_call(
        _post_body,
        out_shape=jax.ShapeDtypeStruct((N, LATENT), jnp.float32),
    )(h, s0, s1, den.reshape(N, 1), bp["gat_bias"].reshape(1, -1),
      bp["Wc"], bp["bc"].reshape(1, -1),
      bp["bn1_g"].reshape(1, -1), bp["bn1_b"].reshape(1, -1),
      bp["Wf1"], bp["bf1"].reshape(1, -1), bp["Wf2"], bp["bf2"].reshape(1, -1),
      bp["bn2_g"].reshape(1, -1), bp["bn2_b"].reshape(1, -1))


def _dec_body(h, wd, bd, out):
    logits = jnp.dot(h[...], wd[...], preferred_element_type=jnp.float32) + bd[...]
    m = jnp.max(logits, axis=-1, keepdims=True)
    e = jnp.exp(logits - m)
    out[...] = e / jnp.sum(e, axis=-1, keepdims=True)


def _decoder(h, p):
    wd = jnp.pad(p["dec_W"], ((0, 0), (0, HALF - NUM_CT)))
    bd = jnp.pad(p["dec_b"], (0, HALF - NUM_CT), constant_values=-1e30)
    probs = pl.pallas_call(
        _dec_body,
        grid=(N // _ROWS,),
        in_specs=[
            pl.BlockSpec((_ROWS, LATENT), lambda i: (i, 0)),
            pl.BlockSpec((LATENT, HALF), lambda i: (0, 0)),
            pl.BlockSpec((1, HALF), lambda i: (0, 0)),
        ],
        out_specs=pl.BlockSpec((_ROWS, HALF), lambda i: (i, 0)),
        out_shape=jax.ShapeDtypeStruct((N, HALF), jnp.float32),
    )(h, wd, bd.reshape(1, -1))
    return probs[:, :NUM_CT]


_CH = 80
_EPT_A = E // 32
_EPT_B = E // 16


def _sc_mesh():
    return plsc.VectorSubcoreMesh(core_axis_name="c", subcore_axis_name="s")


_SC_PARAMS = pltpu.CompilerParams(needs_layout_passes=False)


def _pass_a(xl0, xl1, xr0, xr1, src, dst, ea, we, att):
    @functools.partial(
        pl.kernel,
        out_type=jax.ShapeDtypeStruct((E,), jnp.float32),
        mesh=_sc_mesh(),
        scratch_types=[
            pltpu.VMEM((_CH,), jnp.int32),
            pltpu.VMEM((_CH,), jnp.int32),
            pltpu.VMEM((_CH, HALF), jnp.float32),
            pltpu.VMEM((_CH, HALF), jnp.float32),
            pltpu.VMEM((_CH, HALF), jnp.float32),
            pltpu.VMEM((_CH, HALF), jnp.float32),
            pltpu.VMEM((_CH,), jnp.float32),
            pltpu.VMEM((_CH,), jnp.float32),
            pltpu.VMEM((LATENT,), jnp.float32),
            pltpu.VMEM((LATENT,), jnp.float32),
            pltpu.SemaphoreType.DMA,
        ],
        compiler_params=_SC_PARAMS,
    )
    def k(xl0h, xl1h, xr0h, xr1h, srch, dsth, eah, weh, atth, exh,
          src_v, dst_v, a0, a1, b0, b1, ea_v, ex_v, we_v, att_v, sem):
        cid = lax.axis_index("c")
        sid = lax.axis_index("s")
        wid = sid * 2 + cid
        base = wid * _EPT_A
        pltpu.sync_copy(weh, we_v)
        pltpu.sync_copy(atth, att_v)

        def chunk(j, carry):
            off = base + j * _CH
            pltpu.sync_copy(srch.at[pl.ds(off, _CH)], src_v)
            pltpu.sync_copy(dsth.at[pl.ds(off, _CH)], dst_v)
            pltpu.sync_copy(eah.at[pl.ds(off, _CH)], ea_v)
            cps = [
                pltpu.async_copy(xl0h.at[src_v], a0, sem),
                pltpu.async_copy(xl1h.at[src_v], a1, sem),
                pltpu.async_copy(xr0h.at[dst_v], b0, sem),
                pltpu.async_copy(xr1h.at[dst_v], b1, sem),
            ]
            for cp in cps:
                cp.wait()
            for g in range(_CH // 16):
                e16 = lax.iota(jnp.int32, 16) + (g * 16)
                eav = ea_v[pl.ds(g * 16, 16)]

                def feat(f, acc):
                    fb = jnp.zeros((16,), jnp.int32) + f
                    wef0 = plsc.load_gather(we_v, [fb])
                    atf0 = plsc.load_gather(att_v, [fb])
                    v0 = (plsc.load_gather(a0, [e16, fb])
                          + plsc.load_gather(b0, [e16, fb])
                          + eav * wef0)
                    acc = acc + jnp.maximum(v0, 0.2 * v0) * atf0
                    fb1 = fb + HALF
                    wef1 = plsc.load_gather(we_v, [fb1])
                    atf1 = plsc.load_gather(att_v, [fb1])
                    v1 = (plsc.load_gather(a1, [e16, fb])
                          + plsc.load_gather(b1, [e16, fb])
                          + eav * wef1)
                    return acc + jnp.maximum(v1, 0.2 * v1) * atf1

                acc = lax.fori_loop(0, HALF, feat, jnp.zeros((16,), jnp.float32),
                                    unroll=8)
                ex_v[pl.ds(g * 16, 16)] = jnp.exp(acc)
            pltpu.sync_copy(ex_v, exh.at[pl.ds(off, _CH)])
            return carry

        lax.fori_loop(0, _EPT_A // _CH, chunk, 0)

    return k(xl0, xl1, xr0, xr1, src, dst, ea, we, att)


def _pass_b(xl0, xl1, src, dst, ex):
    oshape = jax.ShapeDtypeStruct((N, HALF), jnp.float32)
    @functools.partial(
        pl.kernel,
        out_type=[oshape, oshape, jax.ShapeDtypeStruct((N,), jnp.float32)],
        mesh=_sc_mesh(),
        scratch_types=[
            pltpu.VMEM((_CH,), jnp.int32),
            pltpu.VMEM((_CH,), jnp.int32),
            pltpu.VMEM((_CH, HALF), jnp.float32),
            pltpu.VMEM((_CH,), jnp.float32),
            pltpu.VMEM((80, HALF), jnp.float32),
            pltpu.VMEM((1008,), jnp.float32),
            pltpu.VMEM_SHARED((N, HALF), jnp.float32),
            pltpu.VMEM_SHARED((N,), jnp.float32),
            pltpu.SemaphoreType.DMA,
        ],
        compiler_params=_SC_PARAMS,
    )
    def k(xl0h, xl1h, srch, dsth, exh, s0h, s1h, denh,
          src_v, dst_v, rows_v, ex_v, zbuf, zden, s_sh, den_sh, sem):
        cid = lax.axis_index("c")
        sid = lax.axis_index("s")
        zv = jnp.zeros((16,), jnp.float32)

        def zrow(i, c):
            for cc in range(HALF // 16):
                zbuf[i, pl.ds(cc * 16, 16)] = zv
            return c

        lax.fori_loop(0, 80, zrow, 0)

        def zel(i, c):
            zden[pl.ds(i * 16, 16)] = zv
            return c

        lax.fori_loop(0, 63, zel, 0)

        for q in range(8):
            bid = sid + q * 16

            @pl.when(bid < 125)
            def _():
                pltpu.sync_copy(zbuf, s_sh.at[pl.ds(bid * 80, 80)])

        @pl.when(sid < 10)
        def _():
            pltpu.sync_copy(zden.at[pl.ds(0, 1000)], den_sh.at[pl.ds(sid * 1000, 1000)])

        plsc.subcore_barrier()

        base = sid * _EPT_B

        def chunk(j, carry):
            off = base + j * _CH
            pltpu.sync_copy(srch.at[pl.ds(off, _CH)], src_v)
            pltpu.sync_copy(dsth.at[pl.ds(off, _CH)], dst_v)
            pltpu.sync_copy(exh.at[pl.ds(off, _CH)], ex_v)

            @pl.when(cid == 0)
            def _():
                pltpu.async_copy(xl0h.at[src_v], rows_v, sem).wait()

            @pl.when(cid == 1)
            def _():
                pltpu.async_copy(xl1h.at[src_v], rows_v, sem).wait()

            def scale(e, c2):
                s = plsc.load_gather(ex_v, [jnp.zeros((16,), jnp.int32) + e])
                for cc in range(HALF // 16):
                    rows_v[e, pl.ds(cc * 16, 16)] = rows_v[e, pl.ds(cc * 16, 16)] * s
                return c2

            lax.fori_loop(0, _CH, scale, 0, unroll=4)
            pltpu.sync_copy(rows_v, s_sh.at[dst_v], add=True)

            @pl.when(cid == 0)
            def _():
                pltpu.sync_copy(ex_v, den_sh.at[dst_v], add=True)

            return carry

        lax.fori_loop(0, _EPT_B // _CH, chunk, 0)
        plsc.subcore_barrier()

        for q in range(8):
            bid = sid + q * 16

            @pl.when((bid < 125) & (cid == 0))
            def _():
                sl = pl.ds(bid * 80, 80)
                pltpu.sync_copy(s_sh.at[sl], zbuf)
                pltpu.sync_copy(zbuf, s0h.at[sl])

            @pl.when((bid < 125) & (cid == 1))
            def _():
                sl = pl.ds(bid * 80, 80)
                pltpu.sync_copy(s_sh.at[sl], zbuf)
                pltpu.sync_copy(zbuf, s1h.at[sl])

        @pl.when((cid == 0) & (sid < 10))
        def _():
            pltpu.sync_copy(den_sh.at[pl.ds(sid * 1000, 1000)],
                            zden.at[pl.ds(0, 1000)])
            pltpu.sync_copy(zden.at[pl.ds(0, 1000)],
                            denh.at[pl.ds(sid * 1000, 1000)])

    return k(xl0, xl1, src, dst, ex)


def kernel(x, pos, edge_attr, edge_index, params):
    src = edge_index[0]
    dst = edge_index[1]
    ea = edge_attr[:, 0]
    xc = jnp.concatenate([x, pos], axis=-1)
    h = _encoder(xc, params)
    for bp in params["blocks"]:
        xl0, xl1, xr0, xr1 = _xlxr(h, bp)
        ex = _pass_a(xl0, xl1, xr0, xr1, src, dst, ea,
                     bp["We"][0], bp["att"])
        s0, s1, den = _pass_b(xl0, xl1, src, dst, ex)
        h = _post(h, s0, s1, den, bp)
    return _decoder(h, params)


# speedup vs baseline: 1.9862x; 1.1293x over previous
import functools

import jax
import jax.numpy as jnp
from jax import lax
from jax.experimental import pallas as pl
from jax.experimental.pallas import tpu as pltpu
from jax.experimental.pallas import tpu_sc as plsc

N = 10000
E = 320000
LATENT = 256
HALF = 128
NUM_CT = 20


_ROWS = 2000


def _enc_body(xc, w1, b1, w2, b2, w3, b3, out):
    h1 = jnp.maximum(jnp.dot(xc[...], w1[...], preferred_element_type=jnp.float32) + b1[...], 0.0)
    h2 = jnp.maximum(jnp.dot(h1, w2[...], preferred_element_type=jnp.float32) + b2[...], 0.0)
    out[...] = jnp.dot(h2, w3[...], preferred_element_type=jnp.float32) + b3[...]


def _encoder(xc, p):
    grid = N // _ROWS
    full = lambda shape: pl.BlockSpec(shape, lambda i: (0, 0))
    return pl.pallas_call(
        _enc_body,
        grid=(grid,),
        in_specs=[
            pl.BlockSpec((_ROWS, 130), lambda i: (i, 0)),
            full((130, 512)), full((1, 512)),
            full((512, 256)), full((1, 256)),
            full((256, LATENT)), full((1, LATENT)),
        ],
        out_specs=pl.BlockSpec((_ROWS, LATENT), lambda i: (i, 0)),
        out_shape=jax.ShapeDtypeStruct((N, LATENT), jnp.float32),
    )(xc, p["mlp_W1"], p["mlp_b1"].reshape(1, -1), p["mlp_W2"],
      p["mlp_b2"].reshape(1, -1), p["mlp_W3"], p["mlp_b3"].reshape(1, -1))


def _xlxr_body(h, wl, bl, wr, br, xlf, xrf, xl0, xl1):
    xl = jnp.dot(h[...], wl[...], preferred_element_type=jnp.float32) + bl[...]
    xr = jnp.dot(h[...], wr[...], preferred_element_type=jnp.float32) + br[...]
    xlf[...] = xl
    xrf[...] = xr
    xl0[...] = xl[:, :HALF]
    xl1[...] = xl[:, HALF:]


def _xlxr(h, bp):
    grid = N // _ROWS
    full = lambda shape: pl.BlockSpec(shape, lambda i: (0, 0))
    hspec = pl.BlockSpec((_ROWS, HALF), lambda i: (i, 0))
    fspec = pl.BlockSpec((_ROWS, LATENT), lambda i: (i, 0))
    hshape = jax.ShapeDtypeStruct((N, HALF), jnp.float32)
    fshape = jax.ShapeDtypeStruct((N, LATENT), jnp.float32)
    return pl.pallas_call(
        _xlxr_body,
        grid=(grid,),
        in_specs=[
            pl.BlockSpec((_ROWS, LATENT), lambda i: (i, 0)),
            full((LATENT, LATENT)), full((1, LATENT)),
            full((LATENT, LATENT)), full((1, LATENT)),
        ],
        out_specs=[fspec, fspec, hspec, hspec],
        out_shape=[fshape, fshape, hshape, hshape],
    )(h, bp["Wl"], bp["bl"].reshape(1, -1), bp["Wr"], bp["br"].reshape(1, -1))


def _post_body(h, s0, s1, den, gb, wc, bc, g1, b1, wf1, bf1, wf2, bf2, g2, b2, out):
    S = jnp.concatenate([s0[...], s1[...]], axis=-1)
    osp = S / (den[...] + 1e-16) + gb[...]
    o = h[...] + jnp.dot(osp, wc[...], preferred_element_type=jnp.float32) + bc[...]
    m1 = jnp.mean(o, axis=0, keepdims=True)
    v1 = jnp.mean((o - m1) * (o - m1), axis=0, keepdims=True)
    o = (o - m1) * jax.lax.rsqrt(v1 + 1e-5) * g1[...] + b1[...]
    hh = jnp.maximum(jnp.dot(o, wf1[...], preferred_element_type=jnp.float32) + bf1[...], 0.0)
    ffn = jnp.dot(hh, wf2[...], preferred_element_type=jnp.float32) + bf2[...]
    o2 = o + ffn
    m2 = jnp.mean(o2, axis=0, keepdims=True)
    v2 = jnp.mean((o2 - m2) * (o2 - m2), axis=0, keepdims=True)
    out[...] = (o2 - m2) * jax.lax.rsqrt(v2 + 1e-5) * g2[...] + b2[...]


def _post(h, s0, s1, den, bp):
    return pl.pallas_call(
        _post_body,
        out_shape=jax.ShapeDtypeStruct((N, LATENT), jnp.float32),
    )(h, s0, s1, den.reshape(N, 1), bp["gat_bias"].reshape(1, -1),
      bp["Wc"], bp["bc"].reshape(1, -1),
      bp["bn1_g"].reshape(1, -1), bp["bn1_b"].reshape(1, -1),
      bp["Wf1"], bp["bf1"].reshape(1, -1), bp["Wf2"], bp["bf2"].reshape(1, -1),
      bp["bn2_g"].reshape(1, -1), bp["bn2_b"].reshape(1, -1))


def _dec_body(h, wd, bd, out):
    logits = jnp.dot(h[...], wd[...], preferred_element_type=jnp.float32) + bd[...]
    m = jnp.max(logits, axis=-1, keepdims=True)
    e = jnp.exp(logits - m)
    out[...] = e / jnp.sum(e, axis=-1, keepdims=True)


def _decoder(h, p):
    wd = jnp.pad(p["dec_W"], ((0, 0), (0, HALF - NUM_CT)))
    bd = jnp.pad(p["dec_b"], (0, HALF - NUM_CT), constant_values=-1e30)
    probs = pl.pallas_call(
        _dec_body,
        grid=(N // _ROWS,),
        in_specs=[
            pl.BlockSpec((_ROWS, LATENT), lambda i: (i, 0)),
            pl.BlockSpec((LATENT, HALF), lambda i: (0, 0)),
            pl.BlockSpec((1, HALF), lambda i: (0, 0)),
        ],
        out_specs=pl.BlockSpec((_ROWS, HALF), lambda i: (i, 0)),
        out_shape=jax.ShapeDtypeStruct((N, HALF), jnp.float32),
    )(h, wd, bd.reshape(1, -1))
    return probs[:, :NUM_CT]


_CH = 80
_EPT_A = E // 32
_EPT_B = E // 16


def _sc_mesh():
    return plsc.VectorSubcoreMesh(core_axis_name="c", subcore_axis_name="s")


_SC_PARAMS = pltpu.CompilerParams(needs_layout_passes=False)


_NCH_A = _EPT_A // _CH


def _pass_a(xl, xr, src, dst, ea, we, att):
    @functools.partial(
        pl.kernel,
        out_type=jax.ShapeDtypeStruct((E,), jnp.float32),
        mesh=_sc_mesh(),
        scratch_types=[
            pltpu.VMEM((_CH,), jnp.int32),
            pltpu.VMEM((_CH,), jnp.int32),
            pltpu.VMEM((_CH,), jnp.float32),
            pltpu.VMEM((_CH,), jnp.int32),
            pltpu.VMEM((_CH,), jnp.int32),
            pltpu.VMEM((_CH,), jnp.float32),
            pltpu.VMEM((_CH, LATENT), jnp.float32),
            pltpu.VMEM((_CH, LATENT), jnp.float32),
            pltpu.VMEM((_CH, LATENT), jnp.float32),
            pltpu.VMEM((_CH, LATENT), jnp.float32),
            pltpu.VMEM((_CH,), jnp.float32),
            pltpu.VMEM((LATENT,), jnp.float32),
            pltpu.VMEM((LATENT,), jnp.float32),
            pltpu.SemaphoreType.DMA,
            pltpu.SemaphoreType.DMA,
        ],
        compiler_params=_SC_PARAMS,
    )
    def k(xlh, xrh, srch, dsth, eah, weh, atth, exh,
          srcA, dstA, eaA, srcB, dstB, eaB, aA, bA, aB, bB,
          ex_v, we_v, att_v, semA, semB):
        cid = lax.axis_index("c")
        sid = lax.axis_index("s")
        wid = sid * 2 + cid
        base = wid * _EPT_A
        pltpu.sync_copy(weh, we_v)
        pltpu.sync_copy(atth, att_v)

        def fire(c, sv, dv, ev, ar, br, sem):
            off = base + c * _CH
            pltpu.sync_copy(srch.at[pl.ds(off, _CH)], sv)
            pltpu.sync_copy(dsth.at[pl.ds(off, _CH)], dv)
            pltpu.sync_copy(eah.at[pl.ds(off, _CH)], ev)
            pltpu.async_copy(xlh.at[sv], ar, sem)
            pltpu.async_copy(xrh.at[dv], br, sem)

        def drain(sv, dv, ar, br, sem):
            pltpu.make_async_copy(xlh.at[sv], ar, sem).wait()
            pltpu.make_async_copy(xrh.at[dv], br, sem).wait()

        def compute(c, ev, ar, br):
            off = base + c * _CH

            def group(g, carry):
                e16 = lax.iota(jnp.int32, 16) + g * 16
                eav = ev[pl.ds(g * 16, 16)]
                accs = [jnp.zeros((16,), jnp.float32) for _ in range(4)]
                for cc in range(LATENT // 16):
                    wch = we_v[pl.ds(cc * 16, 16)]
                    ach = att_v[pl.ds(cc * 16, 16)]
                    for jj in range(16):
                        f = cc * 16 + jj
                        fb = jnp.zeros((16,), jnp.int32) + f
                        wef = jnp.broadcast_to(wch[jj], (16,))
                        atf = jnp.broadcast_to(ach[jj], (16,))
                        m = (plsc.load_gather(ar, [e16, fb])
                             + plsc.load_gather(br, [e16, fb])
                             + eav * wef)
                        accs[f % 4] = accs[f % 4] + jnp.maximum(m, 0.2 * m) * atf
                acc = (accs[0] + accs[1]) + (accs[2] + accs[3])
                ex_v[pl.ds(g * 16, 16)] = jnp.exp(acc)
                return carry

            lax.fori_loop(0, _CH // 16, group, 0)
            pltpu.sync_copy(ex_v, exh.at[pl.ds(off, _CH)])

        fire(0, srcA, dstA, eaA, aA, bA, semA)

        def body2(j, carry):
            c0 = 2 * j
            fire(c0 + 1, srcB, dstB, eaB, aB, bB, semB)
            drain(srcA, dstA, aA, bA, semA)
            compute(c0, eaA, aA, bA)

            @pl.when(c0 + 2 < _NCH_A)
            def _():
                fire(c0 + 2, srcA, dstA, eaA, aA, bA, semA)

            drain(srcB, dstB, aB, bB, semB)
            compute(c0 + 1, eaB, aB, bB)
            return carry

        lax.fori_loop(0, _NCH_A // 2, body2, 0)
        drain(srcA, dstA, aA, bA, semA)
        compute(_NCH_A - 1, eaA, aA, bA)

    return k(xl, xr, src, dst, ea, we, att)


def _pass_b(xl0, xl1, src, dst, ex):
    oshape = jax.ShapeDtypeStruct((N, HALF), jnp.float32)
    @functools.partial(
        pl.kernel,
        out_type=[oshape, oshape, jax.ShapeDtypeStruct((N,), jnp.float32)],
        mesh=_sc_mesh(),
        scratch_types=[
            pltpu.VMEM((_CH,), jnp.int32),
            pltpu.VMEM((_CH,), jnp.int32),
            pltpu.VMEM((_CH, HALF), jnp.float32),
            pltpu.VMEM((_CH,), jnp.float32),
            pltpu.VMEM((80, HALF), jnp.float32),
            pltpu.VMEM((1008,), jnp.float32),
            pltpu.VMEM_SHARED((N, HALF), jnp.float32),
            pltpu.VMEM_SHARED((N,), jnp.float32),
            pltpu.SemaphoreType.DMA,
        ],
        compiler_params=_SC_PARAMS,
    )
    def k(xl0h, xl1h, srch, dsth, exh, s0h, s1h, denh,
          src_v, dst_v, rows_v, ex_v, zbuf, zden, s_sh, den_sh, sem):
        cid = lax.axis_index("c")
        sid = lax.axis_index("s")
        zv = jnp.zeros((16,), jnp.float32)

        def zrow(i, c):
            for cc in range(HALF // 16):
                zbuf[i, pl.ds(cc * 16, 16)] = zv
            return c

        lax.fori_loop(0, 80, zrow, 0)

        def zel(i, c):
            zden[pl.ds(i * 16, 16)] = zv
            return c

        lax.fori_loop(0, 63, zel, 0)

        for q in range(8):
            bid = sid + q * 16

            @pl.when(bid < 125)
            def _():
                pltpu.sync_copy(zbuf, s_sh.at[pl.ds(bid * 80, 80)])

        @pl.when(sid < 10)
        def _():
            pltpu.sync_copy(zden.at[pl.ds(0, 1000)], den_sh.at[pl.ds(sid * 1000, 1000)])

        plsc.subcore_barrier()

        base = sid * _EPT_B

        def chunk(j, carry):
            off = base + j * _CH
            pltpu.sync_copy(srch.at[pl.ds(off, _CH)], src_v)
            pltpu.sync_copy(dsth.at[pl.ds(off, _CH)], dst_v)
            pltpu.sync_copy(exh.at[pl.ds(off, _CH)], ex_v)

            @pl.when(cid == 0)
            def _():
                pltpu.async_copy(xl0h.at[src_v], rows_v, sem).wait()

            @pl.when(cid == 1)
            def _():
                pltpu.async_copy(xl1h.at[src_v], rows_v, sem).wait()

            def scale(e, c2):
                s = plsc.load_gather(ex_v, [jnp.zeros((16,), jnp.int32) + e])
                for cc in range(HALF // 16):
                    rows_v[e, pl.ds(cc * 16, 16)] = rows_v[e, pl.ds(cc * 16, 16)] * s
                return c2

            lax.fori_loop(0, _CH, scale, 0, unroll=4)
            pltpu.sync_copy(rows_v, s_sh.at[dst_v], add=True)

            @pl.when(cid == 0)
            def _():
                pltpu.sync_copy(ex_v, den_sh.at[dst_v], add=True)

            return carry

        lax.fori_loop(0, _EPT_B // _CH, chunk, 0)
        plsc.subcore_barrier()

        for q in range(8):
            bid = sid + q * 16

            @pl.when((bid < 125) & (cid == 0))
            def _():
                sl = pl.ds(bid * 80, 80)
                pltpu.sync_copy(s_sh.at[sl], zbuf)
                pltpu.sync_copy(zbuf, s0h.at[sl])

            @pl.when((bid < 125) & (cid == 1))
            def _():
                sl = pl.ds(bid * 80, 80)
                pltpu.sync_copy(s_sh.at[sl], zbuf)
                pltpu.sync_copy(zbuf, s1h.at[sl])

        @pl.when((cid == 0) & (sid < 10))
        def _():
            pltpu.sync_copy(den_sh.at[pl.ds(sid * 1000, 1000)],
                            zden.at[pl.ds(0, 1000)])
            pltpu.sync_copy(zden.at[pl.ds(0, 1000)],
                            denh.at[pl.ds(sid * 1000, 1000)])

    return k(xl0, xl1, src, dst, ex)


def kernel(x, pos, edge_attr, edge_index, params):
    src = edge_index[0]
    dst = edge_index[1]
    ea = edge_attr[:, 0]
    xc = jnp.concatenate([x, pos], axis=-1)
    h = _encoder(xc, params)
    for bp in params["blocks"]:
        xlf, xrf, xl0, xl1 = _xlxr(h, bp)
        ex = _pass_a(xlf, xrf, src, dst, ea,
                     bp["We"][0], bp["att"])
        s0, s1, den = _pass_b(xl0, xl1, src, dst, ex)
        h = _post(h, s0, s1, den, bp)
    return _decoder(h, params)


# speedup vs baseline: 4.7020x; 2.3673x over previous
import functools

import jax
import jax.numpy as jnp
from jax import lax
from jax.experimental import pallas as pl
from jax.experimental.pallas import tpu as pltpu
from jax.experimental.pallas import tpu_sc as plsc

N = 10000
E = 320000
LATENT = 256
HALF = 128
NUM_CT = 20


_ROWS = 2000


def _enc_body(xc, w1, b1, w2, b2, w3, b3, out):
    h1 = jnp.maximum(jnp.dot(xc[...], w1[...], preferred_element_type=jnp.float32) + b1[...], 0.0)
    h2 = jnp.maximum(jnp.dot(h1, w2[...], preferred_element_type=jnp.float32) + b2[...], 0.0)
    out[...] = jnp.dot(h2, w3[...], preferred_element_type=jnp.float32) + b3[...]


def _encoder(xc, p):
    grid = N // _ROWS
    full = lambda shape: pl.BlockSpec(shape, lambda i: (0, 0))
    return pl.pallas_call(
        _enc_body,
        grid=(grid,),
        in_specs=[
            pl.BlockSpec((_ROWS, 130), lambda i: (i, 0)),
            full((130, 512)), full((1, 512)),
            full((512, 256)), full((1, 256)),
            full((256, LATENT)), full((1, LATENT)),
        ],
        out_specs=pl.BlockSpec((_ROWS, LATENT), lambda i: (i, 0)),
        out_shape=jax.ShapeDtypeStruct((N, LATENT), jnp.float32),
    )(xc, p["mlp_W1"], p["mlp_b1"].reshape(1, -1), p["mlp_W2"],
      p["mlp_b2"].reshape(1, -1), p["mlp_W3"], p["mlp_b3"].reshape(1, -1))


def _xlxr_body(h, wl, bl, wr, br, xlf, xrf, xl0, xl1):
    xl = jnp.dot(h[...], wl[...], preferred_element_type=jnp.float32) + bl[...]
    xr = jnp.dot(h[...], wr[...], preferred_element_type=jnp.float32) + br[...]
    xlf[...] = xl
    xrf[...] = xr
    xl0[...] = xl[:, :HALF]
    xl1[...] = xl[:, HALF:]


def _xlxr(h, bp):
    grid = N // _ROWS
    full = lambda shape: pl.BlockSpec(shape, lambda i: (0, 0))
    hspec = pl.BlockSpec((_ROWS, HALF), lambda i: (i, 0))
    fspec = pl.BlockSpec((_ROWS, LATENT), lambda i: (i, 0))
    hshape = jax.ShapeDtypeStruct((N, HALF), jnp.float32)
    fshape = jax.ShapeDtypeStruct((N, LATENT), jnp.float32)
    return pl.pallas_call(
        _xlxr_body,
        grid=(grid,),
        in_specs=[
            pl.BlockSpec((_ROWS, LATENT), lambda i: (i, 0)),
            full((LATENT, LATENT)), full((1, LATENT)),
            full((LATENT, LATENT)), full((1, LATENT)),
        ],
        out_specs=[fspec, fspec, hspec, hspec],
        out_shape=[fshape, fshape, hshape, hshape],
    )(h, bp["Wl"], bp["bl"].reshape(1, -1), bp["Wr"], bp["br"].reshape(1, -1))


def _post_body(h, s0, s1, den, gb, wc, bc, g1, b1, wf1, bf1, wf2, bf2, g2, b2, out):
    S = jnp.concatenate([s0[...], s1[...]], axis=-1)
    osp = S / (den[...] + 1e-16) + gb[...]
    o = h[...] + jnp.dot(osp, wc[...], preferred_element_type=jnp.float32) + bc[...]
    m1 = jnp.mean(o, axis=0, keepdims=True)
    v1 = jnp.mean((o - m1) * (o - m1), axis=0, keepdims=True)
    o = (o - m1) * jax.lax.rsqrt(v1 + 1e-5) * g1[...] + b1[...]
    hh = jnp.maximum(jnp.dot(o, wf1[...], preferred_element_type=jnp.float32) + bf1[...], 0.0)
    ffn = jnp.dot(hh, wf2[...], preferred_element_type=jnp.float32) + bf2[...]
    o2 = o + ffn
    m2 = jnp.mean(o2, axis=0, keepdims=True)
    v2 = jnp.mean((o2 - m2) * (o2 - m2), axis=0, keepdims=True)
    out[...] = (o2 - m2) * jax.lax.rsqrt(v2 + 1e-5) * g2[...] + b2[...]


def _post(h, s0, s1, den, bp):
    return pl.pallas_call(
        _post_body,
        out_shape=jax.ShapeDtypeStruct((N, LATENT), jnp.float32),
    )(h, s0, s1, den.reshape(N, 1), bp["gat_bias"].reshape(1, -1),
      bp["Wc"], bp["bc"].reshape(1, -1),
      bp["bn1_g"].reshape(1, -1), bp["bn1_b"].reshape(1, -1),
      bp["Wf1"], bp["bf1"].reshape(1, -1), bp["Wf2"], bp["bf2"].reshape(1, -1),
      bp["bn2_g"].reshape(1, -1), bp["bn2_b"].reshape(1, -1))


def _dec_body(h, wd, bd, out):
    logits = jnp.dot(h[...], wd[...], preferred_element_type=jnp.float32) + bd[...]
    m = jnp.max(logits, axis=-1, keepdims=True)
    e = jnp.exp(logits - m)
    out[...] = e / jnp.sum(e, axis=-1, keepdims=True)


def _decoder(h, p):
    wd = jnp.pad(p["dec_W"], ((0, 0), (0, HALF - NUM_CT)))
    bd = jnp.pad(p["dec_b"], (0, HALF - NUM_CT), constant_values=-1e30)
    probs = pl.pallas_call(
        _dec_body,
        grid=(N // _ROWS,),
        in_specs=[
            pl.BlockSpec((_ROWS, LATENT), lambda i: (i, 0)),
            pl.BlockSpec((LATENT, HALF), lambda i: (0, 0)),
            pl.BlockSpec((1, HALF), lambda i: (0, 0)),
        ],
        out_specs=pl.BlockSpec((_ROWS, HALF), lambda i: (i, 0)),
        out_shape=jax.ShapeDtypeStruct((N, HALF), jnp.float32),
    )(h, wd, bd.reshape(1, -1))
    return probs[:, :NUM_CT]


_CH = 80
_EPT_A = E // 32
_EPT_B = E // 16


def _sc_mesh():
    return plsc.VectorSubcoreMesh(core_axis_name="c", subcore_axis_name="s")


_SC_PARAMS = pltpu.CompilerParams(needs_layout_passes=False)


_NCH_A = _EPT_A // _CH


def _pass_a(xl, xr, src, dst, ea, we, att):
    @functools.partial(
        pl.kernel,
        out_type=jax.ShapeDtypeStruct((E,), jnp.float32),
        mesh=_sc_mesh(),
        scratch_types=[
            pltpu.VMEM((_CH,), jnp.int32),
            pltpu.VMEM((_CH,), jnp.int32),
            pltpu.VMEM((_CH,), jnp.float32),
            pltpu.VMEM((_CH,), jnp.int32),
            pltpu.VMEM((_CH,), jnp.int32),
            pltpu.VMEM((_CH,), jnp.float32),
            pltpu.VMEM((_CH, LATENT), jnp.float32),
            pltpu.VMEM((_CH, LATENT), jnp.float32),
            pltpu.VMEM((_CH, LATENT), jnp.float32),
            pltpu.VMEM((_CH, LATENT), jnp.float32),
            pltpu.VMEM((_CH,), jnp.float32),
            pltpu.VMEM((LATENT,), jnp.float32),
            pltpu.VMEM((LATENT,), jnp.float32),
            pltpu.VMEM((16, 17), jnp.float32),
            pltpu.SemaphoreType.DMA,
            pltpu.SemaphoreType.DMA,
        ],
        compiler_params=_SC_PARAMS,
    )
    def k(xlh, xrh, srch, dsth, eah, weh, atth, exh,
          srcA, dstA, eaA, srcB, dstB, eaB, aA, bA, aB, bB,
          ex_v, we_v, att_v, accbuf, semA, semB):
        cid = lax.axis_index("c")
        sid = lax.axis_index("s")
        wid = sid * 2 + cid
        base = wid * _EPT_A
        pltpu.sync_copy(weh, we_v)
        pltpu.sync_copy(atth, att_v)
        WCH = [we_v[pl.ds(i * 16, 16)] for i in range(LATENT // 16)]
        ACH = [att_v[pl.ds(i * 16, 16)] for i in range(LATENT // 16)]

        def fire(c, sv, dv, ev, ar, br, sem):
            off = base + c * _CH
            pltpu.sync_copy(srch.at[pl.ds(off, _CH)], sv)
            pltpu.sync_copy(dsth.at[pl.ds(off, _CH)], dv)
            pltpu.sync_copy(eah.at[pl.ds(off, _CH)], ev)
            pltpu.async_copy(xlh.at[sv], ar, sem)
            pltpu.async_copy(xrh.at[dv], br, sem)

        def drain(sv, dv, ar, br, sem):
            pltpu.make_async_copy(xlh.at[sv], ar, sem).wait()
            pltpu.make_async_copy(xrh.at[dv], br, sem).wait()

        def compute(c, ev, ar, br):
            off = base + c * _CH

            def group(g, carry):
                gb = g * 16
                eag = ev[pl.ds(gb, 16)]
                for e in range(16):
                    row = gb + e
                    eab = jnp.broadcast_to(eag[e], (16,))
                    acc0 = jnp.zeros((16,), jnp.float32)
                    acc1 = jnp.zeros((16,), jnp.float32)
                    for cc in range(LATENT // 16):
                        sl = pl.ds(cc * 16, 16)
                        m = ar[row, sl] + br[row, sl] + eab * WCH[cc]
                        t = jnp.maximum(m, 0.2 * m) * ACH[cc]
                        if cc % 2:
                            acc1 = acc1 + t
                        else:
                            acc0 = acc0 + t
                    accbuf[e, pl.ds(0, 16)] = acc0 + acc1
                e16 = lax.iota(jnp.int32, 16)
                tots = [jnp.zeros((16,), jnp.float32) for _ in range(4)]
                for c2 in range(16):
                    cb = jnp.zeros((16,), jnp.int32) + c2
                    tots[c2 % 4] = tots[c2 % 4] + plsc.load_gather(accbuf, [e16, cb])
                tot = (tots[0] + tots[1]) + (tots[2] + tots[3])
                ex_v[pl.ds(gb, 16)] = jnp.exp(tot)
                return carry

            lax.fori_loop(0, _CH // 16, group, 0)
            pltpu.sync_copy(ex_v, exh.at[pl.ds(off, _CH)])

        fire(0, srcA, dstA, eaA, aA, bA, semA)

        def body2(j, carry):
            c0 = 2 * j
            fire(c0 + 1, srcB, dstB, eaB, aB, bB, semB)
            drain(srcA, dstA, aA, bA, semA)
            compute(c0, eaA, aA, bA)

            @pl.when(c0 + 2 < _NCH_A)
            def _():
                fire(c0 + 2, srcA, dstA, eaA, aA, bA, semA)

            drain(srcB, dstB, aB, bB, semB)
            compute(c0 + 1, eaB, aB, bB)
            return carry

        lax.fori_loop(0, _NCH_A // 2, body2, 0)
        drain(srcA, dstA, aA, bA, semA)
        compute(_NCH_A - 1, eaA, aA, bA)

    return k(xl, xr, src, dst, ea, we, att)


def _pass_b(xl0, xl1, src, dst, ex):
    oshape = jax.ShapeDtypeStruct((N, HALF), jnp.float32)
    @functools.partial(
        pl.kernel,
        out_type=[oshape, oshape, jax.ShapeDtypeStruct((N,), jnp.float32)],
        mesh=_sc_mesh(),
        scratch_types=[
            pltpu.VMEM((_CH,), jnp.int32),
            pltpu.VMEM((_CH,), jnp.int32),
            pltpu.VMEM((_CH, HALF), jnp.float32),
            pltpu.VMEM((_CH,), jnp.float32),
            pltpu.VMEM((80, HALF), jnp.float32),
            pltpu.VMEM((1008,), jnp.float32),
            pltpu.VMEM_SHARED((N, HALF), jnp.float32),
            pltpu.VMEM_SHARED((N,), jnp.float32),
            pltpu.SemaphoreType.DMA,
        ],
        compiler_params=_SC_PARAMS,
    )
    def k(xl0h, xl1h, srch, dsth, exh, s0h, s1h, denh,
          src_v, dst_v, rows_v, ex_v, zbuf, zden, s_sh, den_sh, sem):
        cid = lax.axis_index("c")
        sid = lax.axis_index("s")
        zv = jnp.zeros((16,), jnp.float32)

        def zrow(i, c):
            for cc in range(HALF // 16):
                zbuf[i, pl.ds(cc * 16, 16)] = zv
            return c

        lax.fori_loop(0, 80, zrow, 0)

        def zel(i, c):
            zden[pl.ds(i * 16, 16)] = zv
            return c

        lax.fori_loop(0, 63, zel, 0)

        for q in range(8):
            bid = sid + q * 16

            @pl.when(bid < 125)
            def _():
                pltpu.sync_copy(zbuf, s_sh.at[pl.ds(bid * 80, 80)])

        @pl.when(sid < 10)
        def _():
            pltpu.sync_copy(zden.at[pl.ds(0, 1000)], den_sh.at[pl.ds(sid * 1000, 1000)])

        plsc.subcore_barrier()

        base = sid * _EPT_B

        def chunk(j, carry):
            off = base + j * _CH
            pltpu.sync_copy(srch.at[pl.ds(off, _CH)], src_v)
            pltpu.sync_copy(dsth.at[pl.ds(off, _CH)], dst_v)
            pltpu.sync_copy(exh.at[pl.ds(off, _CH)], ex_v)

            @pl.when(cid == 0)
            def _():
                pltpu.async_copy(xl0h.at[src_v], rows_v, sem).wait()

            @pl.when(cid == 1)
            def _():
                pltpu.async_copy(xl1h.at[src_v], rows_v, sem).wait()

            def scale(e, c2):
                s = plsc.load_gather(ex_v, [jnp.zeros((16,), jnp.int32) + e])
                for cc in range(HALF // 16):
                    rows_v[e, pl.ds(cc * 16, 16)] = rows_v[e, pl.ds(cc * 16, 16)] * s
                return c2

            lax.fori_loop(0, _CH, scale, 0, unroll=4)
            pltpu.sync_copy(rows_v, s_sh.at[dst_v], add=True)

            @pl.when(cid == 0)
            def _():
                pltpu.sync_copy(ex_v, den_sh.at[dst_v], add=True)

            return carry

        lax.fori_loop(0, _EPT_B // _CH, chunk, 0)
        plsc.subcore_barrier()

        for q in range(8):
            bid = sid + q * 16

            @pl.when((bid < 125) & (cid == 0))
            def _():
                sl = pl.ds(bid * 80, 80)
                pltpu.sync_copy(s_sh.at[sl], zbuf)
                pltpu.sync_copy(zbuf, s0h.at[sl])

            @pl.when((bid < 125) & (cid == 1))
            def _():
                sl = pl.ds(bid * 80, 80)
                pltpu.sync_copy(s_sh.at[sl], zbuf)
                pltpu.sync_copy(zbuf, s1h.at[sl])

        @pl.when((cid == 0) & (sid < 10))
        def _():
            pltpu.sync_copy(den_sh.at[pl.ds(sid * 1000, 1000)],
                            zden.at[pl.ds(0, 1000)])
            pltpu.sync_copy(zden.at[pl.ds(0, 1000)],
                            denh.at[pl.ds(sid * 1000, 1000)])

    return k(xl0, xl1, src, dst, ex)


def kernel(x, pos, edge_attr, edge_index, params):
    src = edge_index[0]
    dst = edge_index[1]
    ea = edge_attr[:, 0]
    xc = jnp.concatenate([x, pos], axis=-1)
    h = _encoder(xc, params)
    for bp in params["blocks"]:
        xlf, xrf, xl0, xl1 = _xlxr(h, bp)
        ex = _pass_a(xlf, xrf, src, dst, ea,
                     bp["We"][0], bp["att"])
        s0, s1, den = _pass_b(xl0, xl1, src, dst, ex)
        h = _post(h, s0, s1, den, bp)
    return _decoder(h, params)


# speedup vs baseline: 6.1507x; 1.3081x over previous
import functools

import jax
import jax.numpy as jnp
from jax import lax
from jax.experimental import pallas as pl
from jax.experimental.pallas import tpu as pltpu
from jax.experimental.pallas import tpu_sc as plsc

N = 10000
E = 320000
LATENT = 256
HALF = 128
NUM_CT = 20


_ROWS = 2000


def _enc_body(xc, w1, b1, w2, b2, w3, b3, out):
    h1 = jnp.maximum(jnp.dot(xc[...], w1[...], preferred_element_type=jnp.float32) + b1[...], 0.0)
    h2 = jnp.maximum(jnp.dot(h1, w2[...], preferred_element_type=jnp.float32) + b2[...], 0.0)
    out[...] = jnp.dot(h2, w3[...], preferred_element_type=jnp.float32) + b3[...]


def _encoder(xc, p):
    grid = N // _ROWS
    full = lambda shape: pl.BlockSpec(shape, lambda i: (0, 0))
    return pl.pallas_call(
        _enc_body,
        grid=(grid,),
        in_specs=[
            pl.BlockSpec((_ROWS, 130), lambda i: (i, 0)),
            full((130, 512)), full((1, 512)),
            full((512, 256)), full((1, 256)),
            full((256, LATENT)), full((1, LATENT)),
        ],
        out_specs=pl.BlockSpec((_ROWS, LATENT), lambda i: (i, 0)),
        out_shape=jax.ShapeDtypeStruct((N, LATENT), jnp.float32),
    )(xc, p["mlp_W1"], p["mlp_b1"].reshape(1, -1), p["mlp_W2"],
      p["mlp_b2"].reshape(1, -1), p["mlp_W3"], p["mlp_b3"].reshape(1, -1))


def _xlxr_body(h, wl, bl, wr, br, xlf, xrf, xl0, xl1):
    xl = jnp.dot(h[...], wl[...], preferred_element_type=jnp.float32) + bl[...]
    xr = jnp.dot(h[...], wr[...], preferred_element_type=jnp.float32) + br[...]
    xlf[...] = xl
    xrf[...] = xr
    xl0[...] = xl[:, :HALF]
    xl1[...] = xl[:, HALF:]


def _xlxr(h, bp):
    grid = N // _ROWS
    full = lambda shape: pl.BlockSpec(shape, lambda i: (0, 0))
    hspec = pl.BlockSpec((_ROWS, HALF), lambda i: (i, 0))
    fspec = pl.BlockSpec((_ROWS, LATENT), lambda i: (i, 0))
    hshape = jax.ShapeDtypeStruct((N, HALF), jnp.float32)
    fshape = jax.ShapeDtypeStruct((N, LATENT), jnp.float32)
    return pl.pallas_call(
        _xlxr_body,
        grid=(grid,),
        in_specs=[
            pl.BlockSpec((_ROWS, LATENT), lambda i: (i, 0)),
            full((LATENT, LATENT)), full((1, LATENT)),
            full((LATENT, LATENT)), full((1, LATENT)),
        ],
        out_specs=[fspec, fspec, hspec, hspec],
        out_shape=[fshape, fshape, hshape, hshape],
    )(h, bp["Wl"], bp["bl"].reshape(1, -1), bp["Wr"], bp["br"].reshape(1, -1))


def _post_body(h, s0, s1, den, gb, wc, bc, g1, b1, wf1, bf1, wf2, bf2, g2, b2, out):
    S = jnp.concatenate([s0[...], s1[...]], axis=-1)
    osp = S / (den[...] + 1e-16) + gb[...]
    o = h[...] + jnp.dot(osp, wc[...], preferred_element_type=jnp.float32) + bc[...]
    m1 = jnp.mean(o, axis=0, keepdims=True)
    v1 = jnp.mean((o - m1) * (o - m1), axis=0, keepdims=True)
    o = (o - m1) * jax.lax.rsqrt(v1 + 1e-5) * g1[...] + b1[...]
    hh = jnp.maximum(jnp.dot(o, wf1[...], preferred_element_type=jnp.float32) + bf1[...], 0.0)
    ffn = jnp.dot(hh, wf2[...], preferred_element_type=jnp.float32) + bf2[...]
    o2 = o + ffn
    m2 = jnp.mean(o2, axis=0, keepdims=True)
    v2 = jnp.mean((o2 - m2) * (o2 - m2), axis=0, keepdims=True)
    out[...] = (o2 - m2) * jax.lax.rsqrt(v2 + 1e-5) * g2[...] + b2[...]


def _post(h, s0, s1, den, bp):
    return pl.pallas_call(
        _post_body,
        out_shape=jax.ShapeDtypeStruct((N, LATENT), jnp.float32),
    )(h, s0, s1, den.reshape(N, 1), bp["gat_bias"].reshape(1, -1),
      bp["Wc"], bp["bc"].reshape(1, -1),
      bp["bn1_g"].reshape(1, -1), bp["bn1_b"].reshape(1, -1),
      bp["Wf1"], bp["bf1"].reshape(1, -1), bp["Wf2"], bp["bf2"].reshape(1, -1),
      bp["bn2_g"].reshape(1, -1), bp["bn2_b"].reshape(1, -1))


def _dec_body(h, wd, bd, out):
    logits = jnp.dot(h[...], wd[...], preferred_element_type=jnp.float32) + bd[...]
    m = jnp.max(logits, axis=-1, keepdims=True)
    e = jnp.exp(logits - m)
    out[...] = e / jnp.sum(e, axis=-1, keepdims=True)


def _decoder(h, p):
    wd = jnp.pad(p["dec_W"], ((0, 0), (0, HALF - NUM_CT)))
    bd = jnp.pad(p["dec_b"], (0, HALF - NUM_CT), constant_values=-1e30)
    probs = pl.pallas_call(
        _dec_body,
        grid=(N // _ROWS,),
        in_specs=[
            pl.BlockSpec((_ROWS, LATENT), lambda i: (i, 0)),
            pl.BlockSpec((LATENT, HALF), lambda i: (0, 0)),
            pl.BlockSpec((1, HALF), lambda i: (0, 0)),
        ],
        out_specs=pl.BlockSpec((_ROWS, HALF), lambda i: (i, 0)),
        out_shape=jax.ShapeDtypeStruct((N, HALF), jnp.float32),
    )(h, wd, bd.reshape(1, -1))
    return probs[:, :NUM_CT]


_CH = 80
_EPT_A = E // 32
_EPT_B = E // 16


def _sc_mesh():
    return plsc.VectorSubcoreMesh(core_axis_name="c", subcore_axis_name="s")


_SC_PARAMS = pltpu.CompilerParams(needs_layout_passes=False)


_NCH_A = _EPT_A // _CH


def _pass_a(xl, xr, src, dst, ea, we, att):
    @functools.partial(
        pl.kernel,
        out_type=jax.ShapeDtypeStruct((E,), jnp.float32),
        mesh=_sc_mesh(),
        scratch_types=[
            pltpu.VMEM((_CH,), jnp.int32),
            pltpu.VMEM((_CH,), jnp.int32),
            pltpu.VMEM((_CH,), jnp.float32),
            pltpu.VMEM((_CH,), jnp.int32),
            pltpu.VMEM((_CH,), jnp.int32),
            pltpu.VMEM((_CH,), jnp.float32),
            pltpu.VMEM((_CH, LATENT), jnp.float32),
            pltpu.VMEM((_CH, LATENT), jnp.float32),
            pltpu.VMEM((_CH, LATENT), jnp.float32),
            pltpu.VMEM((_CH, LATENT), jnp.float32),
            pltpu.VMEM((_CH,), jnp.float32),
            pltpu.VMEM((LATENT,), jnp.float32),
            pltpu.VMEM((LATENT,), jnp.float32),
            pltpu.VMEM((16, 17), jnp.float32),
            pltpu.SemaphoreType.DMA,
            pltpu.SemaphoreType.DMA,
        ],
        compiler_params=_SC_PARAMS,
    )
    def k(xlh, xrh, srch, dsth, eah, weh, atth, exh,
          srcA, dstA, eaA, srcB, dstB, eaB, aA, bA, aB, bB,
          ex_v, we_v, att_v, accbuf, semA, semB):
        cid = lax.axis_index("c")
        sid = lax.axis_index("s")
        wid = sid * 2 + cid
        base = wid * _EPT_A
        pltpu.sync_copy(weh, we_v)
        pltpu.sync_copy(atth, att_v)
        WCH = [we_v[pl.ds(i * 16, 16)] for i in range(LATENT // 16)]
        ACH = [att_v[pl.ds(i * 16, 16)] for i in range(LATENT // 16)]

        def fire(c, sv, dv, ev, ar, br, sem):
            off = base + c * _CH
            pltpu.sync_copy(srch.at[pl.ds(off, _CH)], sv)
            pltpu.sync_copy(dsth.at[pl.ds(off, _CH)], dv)
            pltpu.sync_copy(eah.at[pl.ds(off, _CH)], ev)
            pltpu.async_copy(xlh.at[sv], ar, sem)
            pltpu.async_copy(xrh.at[dv], br, sem)

        def drain(sv, dv, ar, br, sem):
            pltpu.make_async_copy(xlh.at[sv], ar, sem).wait()
            pltpu.make_async_copy(xrh.at[dv], br, sem).wait()

        def compute(c, ev, ar, br):
            off = base + c * _CH

            def group(g, carry):
                gb = g * 16
                eag = ev[pl.ds(gb, 16)]
                for e in range(16):
                    row = gb + e
                    eab = jnp.broadcast_to(eag[e], (16,))
                    acc0 = jnp.zeros((16,), jnp.float32)
                    acc1 = jnp.zeros((16,), jnp.float32)
                    for cc in range(LATENT // 16):
                        sl = pl.ds(cc * 16, 16)
                        m = ar[row, sl] + br[row, sl] + eab * WCH[cc]
                        t = jnp.maximum(m, 0.2 * m) * ACH[cc]
                        if cc % 2:
                            acc1 = acc1 + t
                        else:
                            acc0 = acc0 + t
                    accbuf[e, pl.ds(0, 16)] = acc0 + acc1
                e16 = lax.iota(jnp.int32, 16)
                tots = [jnp.zeros((16,), jnp.float32) for _ in range(4)]
                for c2 in range(16):
                    cb = jnp.zeros((16,), jnp.int32) + c2
                    tots[c2 % 4] = tots[c2 % 4] + plsc.load_gather(accbuf, [e16, cb])
                tot = (tots[0] + tots[1]) + (tots[2] + tots[3])
                ex_v[pl.ds(gb, 16)] = jnp.exp(tot)
                return carry

            lax.fori_loop(0, _CH // 16, group, 0)
            pltpu.sync_copy(ex_v, exh.at[pl.ds(off, _CH)])

        fire(0, srcA, dstA, eaA, aA, bA, semA)

        def body2(j, carry):
            c0 = 2 * j
            fire(c0 + 1, srcB, dstB, eaB, aB, bB, semB)
            drain(srcA, dstA, aA, bA, semA)
            compute(c0, eaA, aA, bA)

            @pl.when(c0 + 2 < _NCH_A)
            def _():
                fire(c0 + 2, srcA, dstA, eaA, aA, bA, semA)

            drain(srcB, dstB, aB, bB, semB)
            compute(c0 + 1, eaB, aB, bB)
            return carry

        lax.fori_loop(0, _NCH_A // 2, body2, 0)
        drain(srcA, dstA, aA, bA, semA)
        compute(_NCH_A - 1, eaA, aA, bA)

    return k(xl, xr, src, dst, ea, we, att)


_NCH_B = _EPT_B // _CH


def _pass_b(xl0, xl1, src, dst, ex):
    oshape = jax.ShapeDtypeStruct((N, HALF), jnp.float32)
    @functools.partial(
        pl.kernel,
        out_type=[oshape, oshape, jax.ShapeDtypeStruct((N,), jnp.float32)],
        mesh=_sc_mesh(),
        scratch_types=[
            pltpu.VMEM((_CH,), jnp.int32),
            pltpu.VMEM((_CH,), jnp.int32),
            pltpu.VMEM((_CH,), jnp.float32),
            pltpu.VMEM((_CH,), jnp.int32),
            pltpu.VMEM((_CH,), jnp.int32),
            pltpu.VMEM((_CH,), jnp.float32),
            pltpu.VMEM((_CH, HALF), jnp.float32),
            pltpu.VMEM((_CH, HALF), jnp.float32),
            pltpu.VMEM((80, HALF), jnp.float32),
            pltpu.VMEM((1008,), jnp.float32),
            pltpu.VMEM_SHARED((N, HALF), jnp.float32),
            pltpu.VMEM_SHARED((N,), jnp.float32),
            pltpu.SemaphoreType.DMA,
            pltpu.SemaphoreType.DMA,
            pltpu.SemaphoreType.DMA,
            pltpu.SemaphoreType.DMA,
        ],
        compiler_params=_SC_PARAMS,
    )
    def k(xl0h, xl1h, srch, dsth, exh, s0h, s1h, denh,
          srcA, dstA, exA, srcB, dstB, exB, rowsA, rowsB,
          zbuf, zden, s_sh, den_sh, semGA, semGB, semSA, semSB):
        cid = lax.axis_index("c")
        sid = lax.axis_index("s")
        zv = jnp.zeros((16,), jnp.float32)

        def zrow(i, c):
            for cc in range(HALF // 16):
                zbuf[i, pl.ds(cc * 16, 16)] = zv
            return c

        lax.fori_loop(0, 80, zrow, 0)

        def zel(i, c):
            zden[pl.ds(i * 16, 16)] = zv
            return c

        lax.fori_loop(0, 63, zel, 0)

        for q in range(8):
            bid = sid + q * 16

            @pl.when(bid < 125)
            def _():
                pltpu.sync_copy(zbuf, s_sh.at[pl.ds(bid * 80, 80)])

        @pl.when(sid < 10)
        def _():
            pltpu.sync_copy(zden.at[pl.ds(0, 1000)], den_sh.at[pl.ds(sid * 1000, 1000)])

        plsc.subcore_barrier()

        base = sid * _EPT_B

        def fireG(c, sv, dv, ev, rows, semG):
            off = base + c * _CH
            pltpu.sync_copy(srch.at[pl.ds(off, _CH)], sv)
            pltpu.sync_copy(dsth.at[pl.ds(off, _CH)], dv)
            pltpu.sync_copy(exh.at[pl.ds(off, _CH)], ev)

            @pl.when(cid == 0)
            def _():
                pltpu.async_copy(xl0h.at[sv], rows, semG)

            @pl.when(cid == 1)
            def _():
                pltpu.async_copy(xl1h.at[sv], rows, semG)

        def drainG(sv, rows, semG):
            pltpu.make_async_copy(xl0h.at[sv], rows, semG).wait()

        def scale(rows, ev):
            for g in range(_CH // 16):
                exg = ev[pl.ds(g * 16, 16)]
                for e in range(16):
                    row = g * 16 + e
                    s = jnp.broadcast_to(exg[e], (16,))
                    for cc in range(HALF // 16):
                        sl = pl.ds(cc * 16, 16)
                        rows[row, sl] = rows[row, sl] * s

        def fireS(dv, ev, rows, semS):
            pltpu.async_copy(rows, s_sh.at[dv], semS, add=True)

            @pl.when(cid == 0)
            def _():
                pltpu.async_copy(ev, den_sh.at[dv], semS, add=True)

        def drainS(dv, ev, rows, semS):
            pltpu.make_async_copy(rows, s_sh.at[dv], semS).wait()

            @pl.when(cid == 0)
            def _():
                pltpu.make_async_copy(ev, den_sh.at[dv], semS).wait()

        fireG(0, srcA, dstA, exA, rowsA, semGA)

        def body2(j, carry):
            c0 = 2 * j

            @pl.when(j > 0)
            def _():
                drainS(dstB, exB, rowsB, semSB)

            fireG(c0 + 1, srcB, dstB, exB, rowsB, semGB)
            drainG(srcA, rowsA, semGA)
            scale(rowsA, exA)
            fireS(dstA, exA, rowsA, semSA)
            drainG(srcB, rowsB, semGB)
            scale(rowsB, exB)
            fireS(dstB, exB, rowsB, semSB)
            drainS(dstA, exA, rowsA, semSA)

            @pl.when(c0 + 2 < _NCH_B)
            def _():
                fireG(c0 + 2, srcA, dstA, exA, rowsA, semGA)

            return carry

        lax.fori_loop(0, _NCH_B // 2, body2, 0)
        drainS(dstB, exB, rowsB, semSB)
        plsc.subcore_barrier()

        for q in range(8):
            bid = sid + q * 16

            @pl.when((bid < 125) & (cid == 0))
            def _():
                sl = pl.ds(bid * 80, 80)
                pltpu.sync_copy(s_sh.at[sl], zbuf)
                pltpu.sync_copy(zbuf, s0h.at[sl])

            @pl.when((bid < 125) & (cid == 1))
            def _():
                sl = pl.ds(bid * 80, 80)
                pltpu.sync_copy(s_sh.at[sl], zbuf)
                pltpu.sync_copy(zbuf, s1h.at[sl])

        @pl.when((cid == 0) & (sid < 10))
        def _():
            pltpu.sync_copy(den_sh.at[pl.ds(sid * 1000, 1000)],
                            zden.at[pl.ds(0, 1000)])
            pltpu.sync_copy(zden.at[pl.ds(0, 1000)],
                            denh.at[pl.ds(sid * 1000, 1000)])

    return k(xl0, xl1, src, dst, ex)


def kernel(x, pos, edge_attr, edge_index, params):
    src = edge_index[0]
    dst = edge_index[1]
    ea = edge_attr[:, 0]
    xc = jnp.concatenate([x, pos], axis=-1)
    h = _encoder(xc, params)
    for bp in params["blocks"]:
        xlf, xrf, xl0, xl1 = _xlxr(h, bp)
        ex = _pass_a(xlf, xrf, src, dst, ea,
                     bp["We"][0], bp["att"])
        s0, s1, den = _pass_b(xl0, xl1, src, dst, ex)
        h = _post(h, s0, s1, den, bp)
    return _decoder(h, params)
